# Initial kernel scaffold; baseline (speedup 1.0000x reference)
import jax
import jax.numpy as jnp
from jax.experimental import pallas as pl


def kernel(user_emb_w, item_emb_w, W1, b1, W2, b2, W3, b3, Wp, bp, user_bias_w, item_bias_w, user, item, user_topk, item_topk):
    raise NotImplementedError("write your pallas kernel here")


# speedup vs baseline: 3.0053x; 3.0053x over previous
import functools

import jax
import jax.numpy as jnp
from jax import lax
from jax.experimental import pallas as pl
from jax.experimental.pallas import tpu as pltpu
from jax.experimental.pallas import tpu_sc as plsc

NC = 2
NS = 16
NW = NC * NS
LANES = 16
C = 128
NG = C // LANES


@functools.lru_cache(maxsize=None)
def _build_sc_gather(B, D, K):
    DIN = 2 * (K + 1) * D
    BPW = B // NW
    NCH = BPW // C
    f32, i32 = jnp.float32, jnp.int32
    mesh = plsc.VectorSubcoreMesh(core_axis_name="c", subcore_axis_name="s",
                                  num_cores=NC, num_subcores=NS)

    @functools.partial(
        pl.kernel,
        out_type=(jax.ShapeDtypeStruct((B, DIN), f32),
                  jax.ShapeDtypeStruct((B,), f32)),
        mesh=mesh,
        compiler_params=pltpu.CompilerParams(use_tc_tiling_on_sc=False,
                                             needs_layout_passes=False),
        scratch_types=[
            pltpu.VMEM((2, C), i32),
            pltpu.VMEM((2, K, C), i32),
            pltpu.VMEM((2, K, C), i32),
            pltpu.VMEM((2, C, D), f32),
            pltpu.VMEM((2, K, C, D), f32),
            pltpu.VMEM((2, C), f32),
            pltpu.VMEM((C,), f32),
            pltpu.SemaphoreType.DMA,
            pltpu.SemaphoreType.DMA,
        ],
    )
    def sc_gather(uemb, iemb, utk, itk, ubias, ibias, user, item,
                  x_out, bias_out,
                  qidx, tkidx, nids, selfe, nei, bvals, bsum, semg, semw):
        wid = lax.axis_index("s") * NC + lax.axis_index("c")
        iota = lax.iota(i32, LANES)
        zf = jnp.zeros((LANES,), f32)
        qtab = (user, item)
        etab = (uemb, iemb)
        ttab = (utk, itk)
        btab = (ubias, ibias)

        def chunk_body(t, _):
            base = wid * BPW + t * C
            d0 = [pltpu.async_copy(qtab[s].at[pl.ds(base, C)], qidx.at[s], semg)
                  for s in range(2)]
            for d in d0:
                d.wait()
            d1 = []
            for s in range(2):
                d1.append(pltpu.async_copy(etab[s].at[qidx.at[s]], selfe.at[s], semg))
                d1.append(pltpu.async_copy(btab[s].at[qidx.at[s]], bvals.at[s], semg))
            for s in range(2):
                def g_body(g, _, s=s):
                    u16 = qidx[s, pl.ds(g * LANES, LANES)]
                    u5 = u16 * K
                    for k in range(K):
                        tkidx[s, k, pl.ds(g * LANES, LANES)] = u5 + k
                    return _
                lax.fori_loop(0, NG, g_body, None)
            d2 = []
            for s in range(2):
                for k in range(K):
                    d2.append(pltpu.async_copy(ttab[s].at[tkidx.at[s, k]],
                                               nids.at[s, k], semg))
            for d in d2:
                d.wait()
            d3 = []
            for s in range(2):
                for k in range(K):
                    d3.append(pltpu.async_copy(etab[s].at[nids.at[s, k]],
                                               nei.at[s, k], semg))
            for d in d1:
                d.wait()
            for d in d3:
                d.wait()
            for s in range(2):
                for k in range(K):
                    def f_body(g, _, s=s, k=k):
                        ids16 = nids[s, k, pl.ds(g * LANES, LANES)]
                        mn = jnp.min(ids16)

                        @pl.when(mn == 0)
                        def _fix():
                            z = ids16 == 0
                            rows = g * LANES + iota
                            ssp = jnp.full((LANES,), s, i32)
                            ksp = jnp.full((LANES,), k, i32)
                            for j in range(D):
                                plsc.store_scatter(
                                    nei, [ssp, ksp, rows, jnp.full((LANES,), j, i32)],
                                    zf, mask=z)
                        return _
                    lax.fori_loop(0, NG, f_body, None)
            def b_body(g, _):
                sl = pl.ds(g * LANES, LANES)
                bsum[sl] = bvals[0, sl] + bvals[1, sl]
                return _
            lax.fori_loop(0, NG, b_body, None)
            dw = []
            for s in range(2):
                col0 = s * (DIN // 2)
                dw.append(pltpu.async_copy(
                    selfe.at[s], x_out.at[pl.ds(base, C), pl.ds(col0, D)], semw))
                for k in range(K):
                    dw.append(pltpu.async_copy(
                        nei.at[s, k],
                        x_out.at[pl.ds(base, C), pl.ds(col0 + D + k * D, D)], semw))
            dw.append(pltpu.async_copy(bsum, bias_out.at[pl.ds(base, C)], semw))
            for d in dw:
                d.wait()
            return _

        lax.fori_loop(0, NCH, chunk_body, None)

    return sc_gather


@functools.lru_cache(maxsize=None)
def _build_mlp(B, DIN, H1, H2, H3, bsz=2048):
    f32 = jnp.float32

    def body(x_ref, bias_ref, w1, b1, w2, b2, w3, b3, wp, bp, out_ref):
        x = x_ref[...]
        h = jnp.maximum(jnp.dot(x, w1[...], preferred_element_type=f32)
                        + b1[...][None, :], 0.0)
        h = jnp.maximum(jnp.dot(h, w2[...], preferred_element_type=f32)
                        + b2[...][None, :], 0.0)
        h = jnp.maximum(jnp.dot(h, w3[...], preferred_element_type=f32)
                        + b3[...][None, :], 0.0)
        p = jnp.dot(h, wp[...], preferred_element_type=f32)
        out_ref[...] = p + bp[...][None, :] + bias_ref[...]

    return pl.pallas_call(
        body,
        grid=(B // bsz,),
        in_specs=[
            pl.BlockSpec((bsz, DIN), lambda i: (i, 0)),
            pl.BlockSpec((bsz, 1), lambda i: (i, 0)),
            pl.BlockSpec((DIN, H1), lambda i: (0, 0)),
            pl.BlockSpec((H1,), lambda i: (0,)),
            pl.BlockSpec((H1, H2), lambda i: (0, 0)),
            pl.BlockSpec((H2,), lambda i: (0,)),
            pl.BlockSpec((H2, H3), lambda i: (0, 0)),
            pl.BlockSpec((H3,), lambda i: (0,)),
            pl.BlockSpec((H3, 1), lambda i: (0, 0)),
            pl.BlockSpec((1,), lambda i: (0,)),
        ],
        out_specs=pl.BlockSpec((bsz, 1), lambda i: (i, 0)),
        out_shape=jax.ShapeDtypeStruct((B, 1), f32),
    )


def kernel(user_emb_w, item_emb_w, W1, b1, W2, b2, W3, b3, Wp, bp,
           user_bias_w, item_bias_w, user, item, user_topk, item_topk):
    B = user.shape[0]
    D = user_emb_w.shape[1]
    K = user_topk.shape[1]
    DIN = 2 * (K + 1) * D
    sc_gather = _build_sc_gather(B, D, K)
    x, bias = sc_gather(user_emb_w, item_emb_w,
                        user_topk.reshape(-1), item_topk.reshape(-1),
                        user_bias_w.reshape(-1), item_bias_w.reshape(-1),
                        user, item)
    mlp = _build_mlp(B, DIN, W1.shape[1], W2.shape[1], W3.shape[1])
    pred = mlp(x, bias.reshape(B, 1), W1, b1, W2, b2, W3, b3, Wp, bp)
    return pred[:, 0]


# speedup vs baseline: 3.0063x; 1.0004x over previous
import functools

import jax
import jax.numpy as jnp
from jax import lax
from jax.experimental import pallas as pl
from jax.experimental.pallas import tpu as pltpu
from jax.experimental.pallas import tpu_sc as plsc

NC = 2
NS = 16
NW = NC * NS
LANES = 16
C = 128
NG = C // LANES


@functools.lru_cache(maxsize=None)
def _build_sc_gather(B, D, K):
    DIN = 2 * (K + 1) * D
    BPW = B // NW
    NCH = BPW // C
    f32, i32 = jnp.float32, jnp.int32
    mesh = plsc.VectorSubcoreMesh(core_axis_name="c", subcore_axis_name="s",
                                  num_cores=NC, num_subcores=NS)

    @functools.partial(
        pl.kernel,
        out_type=(jax.ShapeDtypeStruct((B, DIN), f32),
                  jax.ShapeDtypeStruct((B,), f32)),
        mesh=mesh,
        compiler_params=pltpu.CompilerParams(use_tc_tiling_on_sc=False,
                                             needs_layout_passes=False),
        scratch_types=[
            pltpu.VMEM((2, C), i32),
            pltpu.VMEM((2, K, C), i32),
            pltpu.VMEM((2, K, C), i32),
            pltpu.VMEM((2, C, D), f32),
            pltpu.VMEM((2, K, C, D), f32),
            pltpu.VMEM((2, C), f32),
            pltpu.VMEM((C,), f32),
            pltpu.SemaphoreType.DMA,
            pltpu.SemaphoreType.DMA,
        ],
    )
    def sc_gather(uemb, iemb, utk, itk, ubias, ibias, user, item,
                  x_out, bias_out,
                  qidx, tkrows, nids, selfe, nei, bvals, bsum, semg, semw):
        wid = lax.axis_index("s") * NC + lax.axis_index("c")
        iota = lax.iota(i32, LANES)
        zf = jnp.zeros((LANES,), f32)
        qtab = (user, item)
        etab = (uemb, iemb)
        ttab = (utk, itk)
        btab = (ubias, ibias)

        def chunk_body(t, _):
            base = wid * BPW + t * C
            d0 = [pltpu.async_copy(qtab[s].at[pl.ds(base, C)], qidx.at[s], semg)
                  for s in range(2)]
            for d in d0:
                d.wait()
            d1 = []
            for s in range(2):
                d1.append(pltpu.async_copy(etab[s].at[qidx.at[s]], selfe.at[s], semg))
                d1.append(pltpu.async_copy(btab[s].at[qidx.at[s]], bvals.at[s], semg))
            for s in range(2):
                def g_body(g, _, s=s):
                    u16 = qidx[s, pl.ds(g * LANES, LANES)]
                    u5 = u16 * K
                    for k in range(K):
                        tkrows[s, k, pl.ds(g * LANES, LANES)] = u5 + k
                    return _
                lax.fori_loop(0, NG, g_body, None)
            d2 = []
            for s in range(2):
                for k in range(K):
                    d2.append(pltpu.async_copy(ttab[s].at[tkrows.at[s, k]],
                                               nids.at[s, k], semg))
            for d in d2:
                d.wait()
            d3 = []
            for s in range(2):
                for k in range(K):
                    d3.append(pltpu.async_copy(etab[s].at[nids.at[s, k]],
                                               nei.at[s, k], semg))
            for d in d1:
                d.wait()
            for d in d3:
                d.wait()
            for s in range(2):
                for k in range(K):
                    def f_body(g, _, s=s, k=k):
                        ids16 = nids[s, k, pl.ds(g * LANES, LANES)]
                        mn = jnp.min(ids16)

                        @pl.when(mn == 0)
                        def _fix():
                            z = ids16 == 0
                            rows = g * LANES + iota
                            ssp = jnp.full((LANES,), s, i32)
                            ksp = jnp.full((LANES,), k, i32)
                            for j in range(D):
                                plsc.store_scatter(
                                    nei, [ssp, ksp, rows, jnp.full((LANES,), j, i32)],
                                    zf, mask=z)
                        return _
                    lax.fori_loop(0, NG, f_body, None)
            def b_body(g, _):
                sl = pl.ds(g * LANES, LANES)
                bsum[sl] = bvals[0, sl] + bvals[1, sl]
                return _
            lax.fori_loop(0, NG, b_body, None)
            dw = []
            for s in range(2):
                col0 = s * (DIN // 2)
                dw.append(pltpu.async_copy(
                    selfe.at[s], x_out.at[pl.ds(base, C), pl.ds(col0, D)], semw))
                for k in range(K):
                    dw.append(pltpu.async_copy(
                        nei.at[s, k],
                        x_out.at[pl.ds(base, C), pl.ds(col0 + D + k * D, D)], semw))
            dw.append(pltpu.async_copy(bsum, bias_out.at[pl.ds(base, C)], semw))
            for d in dw:
                d.wait()
            return _

        lax.fori_loop(0, NCH, chunk_body, None)

    return sc_gather


@functools.lru_cache(maxsize=None)
def _build_mlp(B, DIN, H1, H2, H3, bsz=2048):
    f32 = jnp.float32

    def body(x_ref, bias_ref, w1, b1, w2, b2, w3, b3, wp, bp, out_ref):
        x = x_ref[...]
        h = jnp.maximum(jnp.dot(x, w1[...], preferred_element_type=f32)
                        + b1[...][None, :], 0.0)
        h = jnp.maximum(jnp.dot(h, w2[...], preferred_element_type=f32)
                        + b2[...][None, :], 0.0)
        h = jnp.maximum(jnp.dot(h, w3[...], preferred_element_type=f32)
                        + b3[...][None, :], 0.0)
        p = jnp.dot(h, wp[...], preferred_element_type=f32)
        out_ref[...] = p + bp[...][None, :] + bias_ref[...]

    return pl.pallas_call(
        body,
        grid=(B // bsz,),
        in_specs=[
            pl.BlockSpec((bsz, DIN), lambda i: (i, 0)),
            pl.BlockSpec((bsz, 1), lambda i: (i, 0)),
            pl.BlockSpec((DIN, H1), lambda i: (0, 0)),
            pl.BlockSpec((H1,), lambda i: (0,)),
            pl.BlockSpec((H1, H2), lambda i: (0, 0)),
            pl.BlockSpec((H2,), lambda i: (0,)),
            pl.BlockSpec((H2, H3), lambda i: (0, 0)),
            pl.BlockSpec((H3,), lambda i: (0,)),
            pl.BlockSpec((H3, 1), lambda i: (0, 0)),
            pl.BlockSpec((1,), lambda i: (0,)),
        ],
        out_specs=pl.BlockSpec((bsz, 1), lambda i: (i, 0)),
        out_shape=jax.ShapeDtypeStruct((B, 1), f32),
    )


def kernel(user_emb_w, item_emb_w, W1, b1, W2, b2, W3, b3, Wp, bp,
           user_bias_w, item_bias_w, user, item, user_topk, item_topk):
    B = user.shape[0]
    D = user_emb_w.shape[1]
    K = user_topk.shape[1]
    DIN = 2 * (K + 1) * D
    sc_gather = _build_sc_gather(B, D, K)
    x, bias = sc_gather(user_emb_w, item_emb_w,
                        user_topk.reshape(-1), item_topk.reshape(-1),
                        user_bias_w.reshape(-1), item_bias_w.reshape(-1),
                        user, item)
    mlp = _build_mlp(B, DIN, W1.shape[1], W2.shape[1], W3.shape[1])
    pred = mlp(x, bias.reshape(B, 1), W1, b1, W2, b2, W3, b3, Wp, bp)
    return pred[:, 0]


# speedup vs baseline: 3.1370x; 1.0434x over previous
import functools

import jax
import jax.numpy as jnp
from jax import lax
from jax.experimental import pallas as pl
from jax.experimental.pallas import tpu as pltpu
from jax.experimental.pallas import tpu_sc as plsc

NC = 2
NS = 16
NW = NC * NS
LANES = 16
C = 128
NG = C // LANES


W_T = 8192
QG = W_T // 4


@functools.lru_cache(maxsize=None)
def _build_transpose(D, N):
    nblk = (N + W_T - 1) // W_T

    def body(t_ref, eye_ref, o_ref):
        t = t_ref[...]
        eye = eye_ref[...]
        parts = [
            lax.dot_general(t[:, q * QG:(q + 1) * QG], eye,
                            (((0,), (0,)), ((), ())),
                            preferred_element_type=jnp.float32)
            for q in range(4)
        ]
        o_ref[...] = jnp.concatenate(parts, axis=1)

    call = pl.pallas_call(
        body,
        grid=(nblk,),
        in_specs=[pl.BlockSpec((D, W_T), lambda i: (0, i)),
                  pl.BlockSpec((D, D), lambda i: (0, 0))],
        out_specs=pl.BlockSpec((QG, 4 * D), lambda i: (i, 0)),
        out_shape=jax.ShapeDtypeStruct((nblk * QG, 4 * D), jnp.float32),
    )

    def run(tT):
        return call(tT, jnp.eye(D, dtype=jnp.float32))

    return run


def _sigma(v):
    return (v & -W_T) + ((v & (QG - 1)) << 2) + ((v >> 11) & 3)


@functools.lru_cache(maxsize=None)
def _build_sc_gather(B, D, K, NU, NI):
    DIN = 2 * (K + 1) * D
    BPW = B // NW
    NCH = BPW // C
    f32, i32 = jnp.float32, jnp.int32
    mesh = plsc.VectorSubcoreMesh(core_axis_name="c", subcore_axis_name="s",
                                  num_cores=NC, num_subcores=NS)

    @functools.partial(
        pl.kernel,
        out_type=(jax.ShapeDtypeStruct((B, DIN), f32),
                  jax.ShapeDtypeStruct((B,), f32)),
        mesh=mesh,
        compiler_params=pltpu.CompilerParams(use_tc_tiling_on_sc=False,
                                             needs_layout_passes=False),
        scratch_types=[
            pltpu.VMEM((2, C), i32),
            pltpu.VMEM((2, C), i32),
            pltpu.VMEM((2, K, C), i32),
            pltpu.VMEM((2, K, C), i32),
            pltpu.VMEM((2, C, D), f32),
            pltpu.VMEM((2, K, C, D), f32),
            pltpu.VMEM((2, C), f32),
            pltpu.VMEM((C,), f32),
            pltpu.SemaphoreType.DMA,
            pltpu.SemaphoreType.DMA,
        ],
    )
    def sc_gather(uemb, iemb, utk, itk, ubias, ibias, user, item,
                  x_out, bias_out,
                  qidx, qsig, tkrows, nids, selfe, nei, bvals, bsum,
                  semg, semw):
        wid = lax.axis_index("s") * NC + lax.axis_index("c")
        iota = lax.iota(i32, LANES)
        zf = jnp.zeros((LANES,), f32)
        qtab = (user, item)
        etab = (uemb, iemb)
        ttab = (utk, itk)
        btab = (ubias, ibias)

        def chunk_body(t, _):
            base = wid * BPW + t * C
            d0 = [pltpu.async_copy(qtab[s].at[pl.ds(base, C)], qidx.at[s], semg)
                  for s in range(2)]
            for d in d0:
                d.wait()
            for s in range(2):
                def s_body(g, _, s=s):
                    sl = pl.ds(g * LANES, LANES)
                    qsig[s, sl] = _sigma(qidx[s, sl])
                    return _
                lax.fori_loop(0, NG, s_body, None)
            d1 = []
            for s in range(2):
                d1.append(pltpu.async_copy(etab[s].at[qidx.at[s]], selfe.at[s], semg))
                d1.append(pltpu.async_copy(btab[s].at[qidx.at[s]], bvals.at[s], semg))
            for s, nrows in ((0, NU), (1, NI)):
                def g_body(g, _, s=s, nrows=nrows):
                    u16 = qidx[s, pl.ds(g * LANES, LANES)]
                    for k in range(K):
                        tkrows[s, k, pl.ds(g * LANES, LANES)] = u16 + k * nrows
                    return _
                lax.fori_loop(0, NG, g_body, None)
            d2 = []
            for s in range(2):
                for k in range(K):
                    d2.append(pltpu.async_copy(ttab[s].at[tkrows.at[s, k]],
                                               nids.at[s, k], semg))
            for d in d2:
                d.wait()
            d3 = []
            for s in range(2):
                for k in range(K):
                    d3.append(pltpu.async_copy(etab[s].at[nids.at[s, k]],
                                               nei.at[s, k], semg))
            for d in d1:
                d.wait()
            for d in d3:
                d.wait()
            for s in range(2):
                for k in range(K):
                    def f_body(g, _, s=s, k=k):
                        ids16 = nids[s, k, pl.ds(g * LANES, LANES)]
                        mn = jnp.min(ids16)

                        @pl.when(mn == 0)
                        def _fix():
                            z = ids16 == 0
                            rows = g * LANES + iota
                            ssp = jnp.full((LANES,), s, i32)
                            ksp = jnp.full((LANES,), k, i32)
                            for j in range(D):
                                plsc.store_scatter(
                                    nei, [ssp, ksp, rows, jnp.full((LANES,), j, i32)],
                                    zf, mask=z)
                        return _
                    lax.fori_loop(0, NG, f_body, None)
            def b_body(g, _):
                sl = pl.ds(g * LANES, LANES)
                bsum[sl] = bvals[0, sl] + bvals[1, sl]
                return _
            lax.fori_loop(0, NG, b_body, None)
            dw = []
            for s in range(2):
                col0 = s * (DIN // 2)
                dw.append(pltpu.async_copy(
                    selfe.at[s], x_out.at[pl.ds(base, C), pl.ds(col0, D)], semw))
                for k in range(K):
                    dw.append(pltpu.async_copy(
                        nei.at[s, k],
                        x_out.at[pl.ds(base, C), pl.ds(col0 + D + k * D, D)], semw))
            dw.append(pltpu.async_copy(bsum, bias_out.at[pl.ds(base, C)], semw))
            for d in dw:
                d.wait()
            return _

        lax.fori_loop(0, NCH, chunk_body, None)

    return sc_gather


@functools.lru_cache(maxsize=None)
def _build_mlp(B, DIN, H1, H2, H3, bsz=2048):
    f32 = jnp.float32

    def body(x_ref, bias_ref, w1, b1, w2, b2, w3, b3, wp, bp, out_ref):
        x = x_ref[...]
        h = jnp.maximum(jnp.dot(x, w1[...], preferred_element_type=f32)
                        + b1[...][None, :], 0.0)
        h = jnp.maximum(jnp.dot(h, w2[...], preferred_element_type=f32)
                        + b2[...][None, :], 0.0)
        h = jnp.maximum(jnp.dot(h, w3[...], preferred_element_type=f32)
                        + b3[...][None, :], 0.0)
        p = jnp.dot(h, wp[...], preferred_element_type=f32)
        out_ref[...] = p + bp[...][None, :] + bias_ref[...]

    return pl.pallas_call(
        body,
        grid=(B // bsz,),
        in_specs=[
            pl.BlockSpec((bsz, DIN), lambda i: (i, 0)),
            pl.BlockSpec((bsz, 1), lambda i: (i, 0)),
            pl.BlockSpec((DIN, H1), lambda i: (0, 0)),
            pl.BlockSpec((H1,), lambda i: (0,)),
            pl.BlockSpec((H1, H2), lambda i: (0, 0)),
            pl.BlockSpec((H2,), lambda i: (0,)),
            pl.BlockSpec((H2, H3), lambda i: (0, 0)),
            pl.BlockSpec((H3,), lambda i: (0,)),
            pl.BlockSpec((H3, 1), lambda i: (0, 0)),
            pl.BlockSpec((1,), lambda i: (0,)),
        ],
        out_specs=pl.BlockSpec((bsz, 1), lambda i: (i, 0)),
        out_shape=jax.ShapeDtypeStruct((B, 1), f32),
    )


def kernel(user_emb_w, item_emb_w, W1, b1, W2, b2, W3, b3, Wp, bp,
           user_bias_w, item_bias_w, user, item, user_topk, item_topk):
    B = user.shape[0]
    D = user_emb_w.shape[1]
    K = user_topk.shape[1]
    DIN = 2 * (K + 1) * D
    NU = user_emb_w.shape[0]
    NI = item_emb_w.shape[0]
    sc_gather = _build_sc_gather(B, D, K, NU, NI)
    x, bias = sc_gather(user_emb_w, item_emb_w,
                        user_topk.T.reshape(-1), item_topk.T.reshape(-1),
                        user_bias_w.reshape(-1), item_bias_w.reshape(-1),
                        user, item)
    mlp = _build_mlp(B, DIN, W1.shape[1], W2.shape[1], W3.shape[1])
    pred = mlp(x, bias.reshape(B, 1), W1, b1, W2, b2, W3, b3, Wp, bp)
    return pred[:, 0]


# speedup vs baseline: 3.6064x; 1.1497x over previous
import functools

import jax
import jax.numpy as jnp
from jax import lax
from jax.experimental import pallas as pl
from jax.experimental.pallas import tpu as pltpu
from jax.experimental.pallas import tpu_sc as plsc

NC = 2
NS = 16
NW = NC * NS
LANES = 16
C = 128
NG = C // LANES


W_T = 8192
QG = W_T // 4


@functools.lru_cache(maxsize=None)
def _build_transpose(D, N):
    nfull = N // W_T
    nblk = nfull + 1

    def body(t_ref, tail_ref, o_ref):
        i = pl.program_id(0)
        t = jnp.where(i < nfull, t_ref[...], tail_ref[...])
        parts = [t[:, q * QG:(q + 1) * QG].T for q in range(4)]
        o_ref[...] = jnp.concatenate(parts, axis=1)

    call = pl.pallas_call(
        body,
        grid=(nblk,),
        in_specs=[
            pl.BlockSpec((D, W_T),
                         lambda i: (0, jnp.minimum(i, nfull - 1))),
            pl.BlockSpec((D, W_T), lambda i: (0, 0)),
        ],
        out_specs=pl.BlockSpec((QG, 4 * D), lambda i: (i, 0)),
        out_shape=jax.ShapeDtypeStruct((nblk * QG, 4 * D), jnp.float32),
    )

    def run(tT):
        tail = jnp.pad(tT[:, nfull * W_T:],
                       ((0, 0), (0, nblk * W_T - N)))
        return call(tT, tail)

    return run


def _sigma(v):
    return (v & -W_T) + ((v & (QG - 1)) << 2) + ((v >> 11) & 3)


@functools.lru_cache(maxsize=None)
def _build_sc_gather(B, D, K, NU, NI):
    DIN = 2 * (K + 1) * D
    BPW = B // NW
    NCH = BPW // C
    f32, i32 = jnp.float32, jnp.int32
    mesh = plsc.VectorSubcoreMesh(core_axis_name="c", subcore_axis_name="s",
                                  num_cores=NC, num_subcores=NS)

    @functools.partial(
        pl.kernel,
        out_type=(jax.ShapeDtypeStruct((DIN // 128, B, 128), f32),
                  jax.ShapeDtypeStruct((B,), f32)),
        mesh=mesh,
        compiler_params=pltpu.CompilerParams(use_tc_tiling_on_sc=False,
                                             needs_layout_passes=False),
        scratch_types=[
            pltpu.VMEM((2, C), i32),
            pltpu.VMEM((2, C), i32),
            pltpu.VMEM((2, K, C), i32),
            pltpu.VMEM((2, K, C), i32),
            pltpu.VMEM((2, C, D), f32),
            pltpu.VMEM((2, K, C, D), f32),
            pltpu.VMEM((2, C), f32),
            pltpu.VMEM((C,), f32),
            pltpu.SemaphoreType.DMA,
            pltpu.SemaphoreType.DMA,
            pltpu.SemaphoreType.DMA,
            pltpu.SemaphoreType.DMA,
            pltpu.SemaphoreType.DMA,
        ],
    )
    def sc_gather(uembp, iembp, utk, itk, ubias, ibias,
                  user, item, x_out, bias_out,
                  qidx, qsig, tkrows, nids, selfe, nei, bvals, bsum,
                  semq, sems, semt, semn, semw):
        wid = lax.axis_index("s") * NC + lax.axis_index("c")
        iota = lax.iota(i32, LANES)
        zf = jnp.zeros((LANES,), f32)
        qtab = (user, item)
        etabp = (uembp, iembp)
        ttab = (utk, itk)
        btab = (ubias, ibias)

        def chunk_body(t, _):
            base = wid * BPW + t * C
            d0 = [pltpu.async_copy(qtab[s].at[pl.ds(base, C)], qidx.at[s], semq)
                  for s in range(2)]
            for d in d0:
                d.wait()
            for s in range(2):
                def s_body(g, _, s=s):
                    sl = pl.ds(g * LANES, LANES)
                    qsig[s, sl] = _sigma(qidx[s, sl])
                    return _
                lax.fori_loop(0, NG, s_body, None)
            d1 = []
            for s in range(2):
                d1.append(pltpu.async_copy(etabp[s].at[qsig.at[s]], selfe.at[s], sems))
                d1.append(pltpu.async_copy(btab[s].at[qidx.at[s]], bvals.at[s], sems))
            for s, nrows in ((0, NU), (1, NI)):
                def g_body(g, _, s=s, nrows=nrows):
                    u16 = qidx[s, pl.ds(g * LANES, LANES)]
                    for k in range(K):
                        tkrows[s, k, pl.ds(g * LANES, LANES)] = u16 + k * nrows
                    return _
                lax.fori_loop(0, NG, g_body, None)
            d2 = []
            for s in range(2):
                for k in range(K):
                    d2.append(pltpu.async_copy(ttab[s].at[tkrows.at[s, k]],
                                               nids.at[s, k], semt))
            for d in d2:
                d.wait()
            for s in range(2):
                for k in range(K):
                    def p_body(g, _, s=s, k=k):
                        sl = pl.ds(g * LANES, LANES)
                        tkrows[s, k, sl] = _sigma(nids[s, k, sl])
                        return _
                    lax.fori_loop(0, NG, p_body, None)
            d3 = []
            for s in range(2):
                for k in range(K):
                    d3.append(pltpu.async_copy(etabp[s].at[tkrows.at[s, k]],
                                               nei.at[s, k], semn))
            for d in d1:
                d.wait()
            for d in d3:
                d.wait()
            for s in range(2):
                for k in range(K):
                    def f_body(g, _, s=s, k=k):
                        ids16 = nids[s, k, pl.ds(g * LANES, LANES)]
                        mn = jnp.min(ids16)

                        @pl.when(mn == 0)
                        def _fix():
                            z = ids16 == 0
                            rows = g * LANES + iota
                            ssp = jnp.full((LANES,), s, i32)
                            ksp = jnp.full((LANES,), k, i32)
                            for j in range(D):
                                plsc.store_scatter(
                                    nei, [ssp, ksp, rows, jnp.full((LANES,), j, i32)],
                                    zf, mask=z)
                        return _
                    lax.fori_loop(0, NG, f_body, None)
            def b_body(g, _):
                sl = pl.ds(g * LANES, LANES)
                bsum[sl] = bvals[0, sl] + bvals[1, sl]
                return _
            lax.fori_loop(0, NG, b_body, None)
            dw = []
            for s in range(2):
                col0 = s * (DIN // 2)
                panels = [(col0, selfe.at[s])]
                panels += [(col0 + D + k * D, nei.at[s, k]) for k in range(K)]
                for c0, src in panels:
                    dw.append(pltpu.async_copy(
                        src,
                        x_out.at[c0 // 128, pl.ds(base, C), pl.ds(c0 % 128, D)],
                        semw))
            dw.append(pltpu.async_copy(bsum, bias_out.at[pl.ds(base, C)], semw))
            for d in dw:
                d.wait()
            return _

        lax.fori_loop(0, NCH, chunk_body, None)

    return sc_gather


@functools.lru_cache(maxsize=None)
def _build_mlp(B, DIN, H1, H2, H3, bsz=2048):
    f32 = jnp.float32

    def body(x3_ref, bias_ref, w1, b1, w2, b2, w3, b3, wp, bp, out_ref):
        x3 = x3_ref[...]
        x = jnp.concatenate([x3[j] for j in range(DIN // 128)], axis=1)
        h = jnp.maximum(jnp.dot(x, w1[...], preferred_element_type=f32)
                        + b1[...][None, :], 0.0)
        h = jnp.maximum(jnp.dot(h, w2[...], preferred_element_type=f32)
                        + b2[...][None, :], 0.0)
        h = jnp.maximum(jnp.dot(h, w3[...], preferred_element_type=f32)
                        + b3[...][None, :], 0.0)
        p = jnp.dot(h, wp[...], preferred_element_type=f32)
        out_ref[...] = p + bp[...][None, :] + bias_ref[...]

    return pl.pallas_call(
        body,
        grid=(B // bsz,),
        in_specs=[
            pl.BlockSpec((DIN // 128, bsz, 128), lambda i: (0, i, 0)),
            pl.BlockSpec((bsz, 1), lambda i: (i, 0)),
            pl.BlockSpec((DIN, H1), lambda i: (0, 0)),
            pl.BlockSpec((H1,), lambda i: (0,)),
            pl.BlockSpec((H1, H2), lambda i: (0, 0)),
            pl.BlockSpec((H2,), lambda i: (0,)),
            pl.BlockSpec((H2, H3), lambda i: (0, 0)),
            pl.BlockSpec((H3,), lambda i: (0,)),
            pl.BlockSpec((H3, 1), lambda i: (0, 0)),
            pl.BlockSpec((1,), lambda i: (0,)),
        ],
        out_specs=pl.BlockSpec((bsz, 1), lambda i: (i, 0)),
        out_shape=jax.ShapeDtypeStruct((B, 1), f32),
    )


def kernel(user_emb_w, item_emb_w, W1, b1, W2, b2, W3, b3, Wp, bp,
           user_bias_w, item_bias_w, user, item, user_topk, item_topk):
    B = user.shape[0]
    D = user_emb_w.shape[1]
    K = user_topk.shape[1]
    DIN = 2 * (K + 1) * D
    NU = user_emb_w.shape[0]
    NI = item_emb_w.shape[0]
    tu = _build_transpose(D, NU)(user_emb_w.T)
    ti = _build_transpose(D, NI)(item_emb_w.T)
    uemb_rm = tu.reshape(tu.shape[0] * 4, D)
    iemb_rm = ti.reshape(ti.shape[0] * 4, D)
    sc_gather = _build_sc_gather(B, D, K, NU, NI)
    x, bias = sc_gather(uemb_rm, iemb_rm,
                        user_topk.T.reshape(-1), item_topk.T.reshape(-1),
                        user_bias_w.reshape(-1), item_bias_w.reshape(-1),
                        user, item)
    mlp = _build_mlp(B, DIN, W1.shape[1], W2.shape[1], W3.shape[1])
    pred = mlp(x, bias.reshape(B, 1), W1, b1, W2, b2, W3, b3, Wp, bp)
    return pred[:, 0]
